# Initial kernel scaffold; baseline (speedup 1.0000x reference)
import jax
import jax.numpy as jnp
from jax.experimental import pallas as pl


def kernel(xyzt, prob_table, table3d, table4d, pW1, pW2, pW3, sW1, sW2, sW3, dW1, dW2, dW3):
    raise NotImplementedError("write your pallas kernel here")


# speedup vs baseline: 29.6959x; 29.6959x over previous
import numpy as np
import jax
import jax.numpy as jnp
from jax import lax
from jax.experimental import pallas as pl
from jax.experimental.pallas import tpu as pltpu
from jax.experimental.pallas import tpu_sc as plsc

_N = 262144
_LOG2_T = 19
_T = 1 << _LOG2_T
_MASK = np.uint32(_T - 1)
_PRIMES = (np.uint32(1), np.uint32(2654435761), np.uint32(805459861),
           np.uint32(3674653429))
_BASE_RES = 16
_PROB_LEVELS = 8
_L3D = 16
_L4D = 16

_NC, _NS = 2, 16
_NW = _NC * _NS
_B = 128
_G = _B // 16
_CHUNKS = _N // (_NW * _B)


def _sc_encode_body(xyztT, ptbl, stbl, dtbl,
                    encP, encS, encD,
                    xcols, fracb, idx3, sub3, rows3, idx4, sub4, rows4,
                    encPb, encSb, encDb, sem):
    wid = lax.axis_index("s") * _NC + lax.axis_index("c")
    iota16 = lax.iota(jnp.int32, 16)
    zero16 = jnp.zeros((16,), jnp.int32)
    one16 = jnp.full((16,), 1, jnp.int32)

    def encode_level(l, scalev, d, tbl_ref, encb, idxb, subb, rowsb):
        ncorner = 1 << d
        resv = (jnp.float32(_BASE_RES) * scalev).astype(jnp.int32).astype(jnp.float32)
        row0 = l * _T

        def idx_group(g, carry):
            s0 = g * 16
            hs = []
            for dd in range(d):
                x16 = xcols[dd, pl.ds(s0, 16)]
                pos = x16 * resv
                c0i = pos.astype(jnp.int32)
                frac = pos - c0i.astype(jnp.float32)
                fracb[dd, pl.ds(s0, 16)] = frac
                c0u = plsc.bitcast(c0i, jnp.uint32)
                h0 = c0u * _PRIMES[dd] if dd else c0u
                h1 = h0 + _PRIMES[dd]
                hs.append((h0, h1))
            for c in range(ncorner):
                h = hs[0][c & 1]
                for dd in range(1, d):
                    h = h ^ hs[dd][(c >> dd) & 1]
                full = plsc.bitcast(h & _MASK, jnp.int32) + row0
                idxb[pl.ds(c * _B + s0, 16)] = lax.shift_right_logical(full, 2)
                subb[pl.ds(c * _B + s0, 16)] = (full & 3) * 2
            return carry

        lax.fori_loop(0, _G, idx_group, None)

        pltpu.async_copy(tbl_ref.at[idxb], rowsb, sem).wait()

        def acc_group(g, carry):
            s0 = g * 16
            svec = s0 + iota16
            fg = []
            for dd in range(d):
                f = fracb[dd, pl.ds(s0, 16)]
                fg.append((1.0 - f, f))
            acc0 = jnp.zeros((16,), jnp.float32)
            acc1 = jnp.zeros((16,), jnp.float32)
            for c in range(ncorner):
                w = fg[0][c & 1]
                for dd in range(1, d):
                    w = w * fg[dd][(c >> dd) & 1]
                rvec = c * _B + svec
                sub = subb[pl.ds(c * _B + s0, 16)]
                f0 = plsc.load_gather(rowsb, [rvec, sub])
                f1 = plsc.load_gather(rowsb, [rvec, sub + 1])
                acc0 = acc0 + w * f0
                acc1 = acc1 + w * f1
            colv = jnp.full((16,), 2 * l, jnp.int32)
            plsc.store_scatter(encb, [svec, colv], acc0)
            plsc.store_scatter(encb, [svec, colv + 1], acc1)
            return carry

        lax.fori_loop(0, _G, acc_group, None)

    ones_f = jnp.ones((16,), jnp.float32)

    def chunk(i, carry):
        base = wid * (_B * _CHUNKS) + i * _B
        pltpu.sync_copy(xyztT.at[:, pl.ds(base, _B)], xcols)

        def level_body(d, tbl_ref, encb, scale, idxb, subb, rowsb):
            def f(l, sv):
                encode_level(l, sv, d, tbl_ref, encb, idxb, subb, rowsb)
                return sv * jnp.float32(scale)
            return f

        lax.fori_loop(0, _PROB_LEVELS,
                      level_body(3, ptbl, encPb, 1.5, idx3, sub3, rows3),
                      ones_f)
        pltpu.sync_copy(encPb, encP.at[pl.ds(base, _B)])

        lax.fori_loop(0, _L3D,
                      level_body(3, stbl, encSb, 1.3819, idx3, sub3, rows3),
                      ones_f)
        pltpu.sync_copy(encSb, encS.at[pl.ds(base, _B)])

        lax.fori_loop(0, _L4D,
                      level_body(4, dtbl, encDb, 1.3819, idx4, sub4, rows4),
                      ones_f)
        pltpu.sync_copy(encDb, encD.at[pl.ds(base, _B)])
        return carry

    lax.fori_loop(0, _CHUNKS, chunk, None)


def _sc_encode(xyztT, ptbl, stbl, dtbl):
    mesh = plsc.VectorSubcoreMesh(core_axis_name="c", subcore_axis_name="s")
    f32 = jnp.float32
    return pl.kernel(
        _sc_encode_body,
        out_type=[
            jax.ShapeDtypeStruct((_N, 2 * _PROB_LEVELS), f32),
            jax.ShapeDtypeStruct((_N, 2 * _L3D), f32),
            jax.ShapeDtypeStruct((_N, 2 * _L4D), f32),
        ],
        mesh=mesh,
        compiler_params=pltpu.CompilerParams(needs_layout_passes=False,
                                             use_tc_tiling_on_sc=False),
        scratch_types=[
            pltpu.VMEM((4, _B), f32),
            pltpu.VMEM((4, _B), f32),
            pltpu.VMEM((8 * _B,), jnp.int32),
            pltpu.VMEM((8 * _B,), jnp.int32),
            pltpu.VMEM((8 * _B, 8), f32),
            pltpu.VMEM((16 * _B,), jnp.int32),
            pltpu.VMEM((16 * _B,), jnp.int32),
            pltpu.VMEM((16 * _B, 8), f32),
            pltpu.VMEM((_B, 2 * _PROB_LEVELS), f32),
            pltpu.VMEM((_B, 2 * _L3D), f32),
            pltpu.VMEM((_B, 2 * _L4D), f32),
            pltpu.SemaphoreType.DMA,
        ],
    )(xyztT, ptbl, stbl, dtbl)


_BLK = 2048


def _mlp_gate_body(encP, encS, encD, pW1, pW2, pW3, sW1, sW2, sW3,
                   dW1, dW2, dW3, o_p, o_s, o_d, o_gs, o_gd, o_w):
    def mm(a, b):
        return lax.dot_general(a, b, (((1,), (0,)), ((), ())),
                               precision=lax.Precision.HIGHEST,
                               preferred_element_type=jnp.float32)

    def mlp(e, W1, W2, W3):
        h = jnp.maximum(mm(e, W1[...]), 0.0)
        h = jnp.maximum(mm(h, W2[...]), 0.0)
        return mm(h, W3[...])

    p = mlp(encP[...], pW1, pW2, pW3)
    s = mlp(encS[...], sW1, sW2, sW3)
    d = mlp(encD[...], dW1, dW2, dW3)
    gs = (1.0 - p) * s
    gd = p * d
    o_p[...] = p
    o_s[...] = s
    o_d[...] = d
    o_gs[...] = gs
    o_gd[...] = gd
    o_w[...] = gs + gd


def _mlp_gate(encP, encS, encD, weights):
    f32 = jnp.float32
    full = lambda shape: pl.BlockSpec(shape, lambda i: (0, 0))
    out_spec = pl.BlockSpec((_BLK, 1), lambda i: (i, 0))
    return pl.pallas_call(
        _mlp_gate_body,
        grid=(_N // _BLK,),
        in_specs=[
            pl.BlockSpec((_BLK, 2 * _PROB_LEVELS), lambda i: (i, 0)),
            pl.BlockSpec((_BLK, 2 * _L3D), lambda i: (i, 0)),
            pl.BlockSpec((_BLK, 2 * _L4D), lambda i: (i, 0)),
            full((2 * _PROB_LEVELS, 64)), full((64, 64)), full((64, 1)),
            full((2 * _L3D, 64)), full((64, 64)), full((64, 1)),
            full((2 * _L4D, 64)), full((64, 64)), full((64, 1)),
        ],
        out_specs=[out_spec] * 6,
        out_shape=[jax.ShapeDtypeStruct((_N, 1), f32)] * 6,
    )(encP, encS, encD, *weights)


def kernel(xyzt, prob_table, table3d, table4d,
           pW1, pW2, pW3, sW1, sW2, sW3, dW1, dW2, dW3):
    xyztT = xyzt.T
    ptbl = prob_table.reshape(_PROB_LEVELS * _T // 4, 8)
    stbl = table3d.reshape(_L3D * _T // 4, 8)
    dtbl = table4d.reshape(_L4D * _T // 4, 8)
    encP, encS, encD = _sc_encode(xyztT, ptbl, stbl, dtbl)
    return _mlp_gate(encP, encS, encD,
                     (pW1, pW2, pW3, sW1, sW2, sW3, dW1, dW2, dW3))


# speedup vs baseline: 68.4963x; 2.3066x over previous
import numpy as np
import jax
import jax.numpy as jnp
from jax import lax
from jax.experimental import pallas as pl
from jax.experimental.pallas import tpu as pltpu
from jax.experimental.pallas import tpu_sc as plsc

_N = 262144
_LOG2_T = 19
_T = 1 << _LOG2_T
_MASK = np.uint32(_T - 1)
_PRIMES = (np.uint32(1), np.uint32(2654435761), np.uint32(805459861),
           np.uint32(3674653429))
_BASE_RES = 16
_PROB_LEVELS = 8
_L3D = 16
_L4D = 16

_NC, _NS = 2, 16
_NW = _NC * _NS
_B = 128
_G = _B // 16
_CHUNKS = _N // (_NW * _B)


def _sc_encode_body(xyztT, ptbl, stbl, dtbl,
                    encP, encS, encD,
                    xcols, fracb, idx3, sub3, rows3, idx4, sub4, rows4,
                    encPb, encSb, encDb, sem):
    wid = lax.axis_index("s") * _NC + lax.axis_index("c")
    iota16 = lax.iota(jnp.int32, 16)
    zero16 = jnp.zeros((16,), jnp.int32)
    one16 = jnp.full((16,), 1, jnp.int32)

    def encode_level(l, scalev, d, tbl_ref, encb, idxb, subb, rowsb):
        ncorner = 1 << d
        resv = (jnp.float32(_BASE_RES) * scalev).astype(jnp.int32).astype(jnp.float32)
        row0 = l * (_T // 4)

        def idx_group(g, carry):
            s0 = g * 16
            hs = []
            for dd in range(d):
                x16 = xcols[dd, pl.ds(s0, 16)]
                pos = x16 * resv
                c0i = pos.astype(jnp.int32)
                frac = pos - c0i.astype(jnp.float32)
                fracb[dd, pl.ds(s0, 16)] = frac
                c0u = plsc.bitcast(c0i, jnp.uint32)
                h0 = c0u * _PRIMES[dd] if dd else c0u
                h1 = h0 + _PRIMES[dd]
                hs.append((h0, h1))
            for c in range(ncorner):
                h = hs[0][c & 1]
                for dd in range(1, d):
                    h = h ^ hs[dd][(c >> dd) & 1]
                t = plsc.bitcast(h & _MASK, jnp.int32)
                r0 = (row0 + lax.shift_right_logical(t, 7) * 32
                      + (lax.shift_right_logical(t, 3) & 15))
                idxb[pl.ds(2 * c * _B + s0, 16)] = r0
                idxb[pl.ds((2 * c + 1) * _B + s0, 16)] = r0 + 16
                subb[pl.ds(c * _B + s0, 16)] = t & 7
            return carry

        lax.fori_loop(0, _G, idx_group, None)

        pltpu.async_copy(tbl_ref.at[idxb], rowsb, sem).wait()

        def acc_group(g, carry):
            s0 = g * 16
            svec = s0 + iota16
            fg = []
            for dd in range(d):
                f = fracb[dd, pl.ds(s0, 16)]
                fg.append((1.0 - f, f))
            acc0 = jnp.zeros((16,), jnp.float32)
            acc1 = jnp.zeros((16,), jnp.float32)
            for c in range(ncorner):
                w = fg[0][c & 1]
                for dd in range(1, d):
                    w = w * fg[dd][(c >> dd) & 1]
                sub = subb[pl.ds(c * _B + s0, 16)]
                f0 = plsc.load_gather(rowsb, [2 * c * _B + svec, sub])
                f1 = plsc.load_gather(rowsb, [(2 * c + 1) * _B + svec, sub])
                acc0 = acc0 + w * f0
                acc1 = acc1 + w * f1
            colv = jnp.full((16,), 2 * l, jnp.int32)
            plsc.store_scatter(encb, [svec, colv], acc0)
            plsc.store_scatter(encb, [svec, colv + 1], acc1)
            return carry

        lax.fori_loop(0, _G, acc_group, None)

    ones_f = jnp.ones((16,), jnp.float32)

    def chunk(i, carry):
        base = wid * (_B * _CHUNKS) + i * _B
        pltpu.sync_copy(xyztT.at[wid * _CHUNKS + i], xcols)

        def level_body(d, tbl_ref, encb, scale, idxb, subb, rowsb):
            def f(l, sv):
                encode_level(l, sv, d, tbl_ref, encb, idxb, subb, rowsb)
                return sv * jnp.float32(scale)
            return f

        lax.fori_loop(0, _PROB_LEVELS,
                      level_body(3, ptbl, encPb, 1.5, idx3, sub3, rows3),
                      ones_f)
        pltpu.sync_copy(encPb, encP.at[pl.ds(base, _B)])

        lax.fori_loop(0, _L3D,
                      level_body(3, stbl, encSb, 1.3819, idx3, sub3, rows3),
                      ones_f)
        pltpu.sync_copy(encSb, encS.at[pl.ds(base, _B)])

        lax.fori_loop(0, _L4D,
                      level_body(4, dtbl, encDb, 1.3819, idx4, sub4, rows4),
                      ones_f)
        pltpu.sync_copy(encDb, encD.at[pl.ds(base, _B)])
        return carry

    lax.fori_loop(0, _CHUNKS, chunk, None)


def _sc_encode(xyztT, ptbl, stbl, dtbl):
    mesh = plsc.VectorSubcoreMesh(core_axis_name="c", subcore_axis_name="s")
    f32 = jnp.float32
    return pl.kernel(
        _sc_encode_body,
        out_type=[
            jax.ShapeDtypeStruct((_N, 2 * _PROB_LEVELS), f32),
            jax.ShapeDtypeStruct((_N, 2 * _L3D), f32),
            jax.ShapeDtypeStruct((_N, 2 * _L4D), f32),
        ],
        mesh=mesh,
        compiler_params=pltpu.CompilerParams(needs_layout_passes=False,
                                             use_tc_tiling_on_sc=False),
        scratch_types=[
            pltpu.VMEM((4, _B), f32),
            pltpu.VMEM((4, _B), f32),
            pltpu.VMEM((16 * _B,), jnp.int32),
            pltpu.VMEM((8 * _B,), jnp.int32),
            pltpu.VMEM((16 * _B, 8), f32),
            pltpu.VMEM((32 * _B,), jnp.int32),
            pltpu.VMEM((16 * _B,), jnp.int32),
            pltpu.VMEM((32 * _B, 8), f32),
            pltpu.VMEM((_B, 2 * _PROB_LEVELS), f32),
            pltpu.VMEM((_B, 2 * _L3D), f32),
            pltpu.VMEM((_B, 2 * _L4D), f32),
            pltpu.SemaphoreType.DMA,
        ],
    )(xyztT, ptbl, stbl, dtbl)


_BLK = 2048


def _mlp_gate_body(encP, encS, encD, pW1, pW2, pW3, sW1, sW2, sW3,
                   dW1, dW2, dW3, o_p, o_s, o_d, o_gs, o_gd, o_w):
    def mm(a, b):
        return lax.dot_general(a, b, (((1,), (0,)), ((), ())),
                               precision=lax.Precision.HIGHEST,
                               preferred_element_type=jnp.float32)

    def mlp(e, W1, W2, W3):
        h = jnp.maximum(mm(e, W1[...]), 0.0)
        h = jnp.maximum(mm(h, W2[...]), 0.0)
        return mm(h, W3[...])

    p = mlp(encP[...], pW1, pW2, pW3)
    s = mlp(encS[...], sW1, sW2, sW3)
    d = mlp(encD[...], dW1, dW2, dW3)
    gs = (1.0 - p) * s
    gd = p * d
    o_p[...] = p
    o_s[...] = s
    o_d[...] = d
    o_gs[...] = gs
    o_gd[...] = gd
    o_w[...] = gs + gd


def _mlp_gate(encP, encS, encD, weights):
    f32 = jnp.float32
    full = lambda shape: pl.BlockSpec(shape, lambda i: (0, 0))
    out_spec = pl.BlockSpec((_BLK, 1), lambda i: (i, 0))
    return pl.pallas_call(
        _mlp_gate_body,
        grid=(_N // _BLK,),
        in_specs=[
            pl.BlockSpec((_BLK, 2 * _PROB_LEVELS), lambda i: (i, 0)),
            pl.BlockSpec((_BLK, 2 * _L3D), lambda i: (i, 0)),
            pl.BlockSpec((_BLK, 2 * _L4D), lambda i: (i, 0)),
            full((2 * _PROB_LEVELS, 64)), full((64, 64)), full((64, 1)),
            full((2 * _L3D, 64)), full((64, 64)), full((64, 1)),
            full((2 * _L4D, 64)), full((64, 64)), full((64, 1)),
        ],
        out_specs=[out_spec] * 6,
        out_shape=[jax.ShapeDtypeStruct((_N, 1), f32)] * 6,
    )(encP, encS, encD, *weights)


def kernel(xyzt, prob_table, table3d, table4d,
           pW1, pW2, pW3, sW1, sW2, sW3, dW1, dW2, dW3):
    xyztT = xyzt.reshape(_N // 128, 128, 4).swapaxes(1, 2)

    def tview(tbl, L):
        return (tbl.reshape(L, _T // 128, 128, 2).swapaxes(2, 3)
                .reshape(L * _T // 4, 8))

    ptbl = tview(prob_table, _PROB_LEVELS)
    stbl = tview(table3d, _L3D)
    dtbl = tview(table4d, _L4D)
    encP, encS, encD = _sc_encode(xyztT, ptbl, stbl, dtbl)
    return _mlp_gate(encP, encS, encD,
                     (pW1, pW2, pW3, sW1, sW2, sW3, dW1, dW2, dW3))


# speedup vs baseline: 90.7738x; 1.3252x over previous
import numpy as np
import jax
import jax.numpy as jnp
from jax import lax
from jax.experimental import pallas as pl
from jax.experimental.pallas import tpu as pltpu
from jax.experimental.pallas import tpu_sc as plsc

_N = 262144
_LOG2_T = 19
_T = 1 << _LOG2_T
_MASK = np.uint32(_T - 1)
_PRIMES = (np.uint32(1), np.uint32(2654435761), np.uint32(805459861),
           np.uint32(3674653429))
_BASE_RES = 16
_PROB_LEVELS = 8
_L3D = 16
_L4D = 16

_NC, _NS = 2, 16
_NW = _NC * _NS
_B = 128
_G = _B // 16
_CHUNKS = _N // (_NW * _B)


def _sc_encode_body(xyztT, ptbl, stbl, dtbl,
                    encP, encS, encD,
                    xcols,
                    frac3a, idx3a, sub3a, rows3a,
                    frac3b, idx3b, sub3b, rows3b,
                    frac4a, idx4a, sub4a, rows4a,
                    frac4b, idx4b, sub4b, rows4b,
                    encPb, encSb, encDb, semA, semB):
    wid = lax.axis_index("s") * _NC + lax.axis_index("c")
    iota16 = lax.iota(jnp.int32, 16)

    set3a = (frac3a, idx3a, sub3a, rows3a, semA)
    set3b = (frac3b, idx3b, sub3b, rows3b, semB)
    set4a = (frac4a, idx4a, sub4a, rows4a, semA)
    set4b = (frac4b, idx4b, sub4b, rows4b, semB)

    def idx_phase(l, scalev, d, tbl_ref, bufs):
        fracb, idxb, subb, rowsb, sem = bufs
        ncorner = 1 << d
        resv = (jnp.float32(_BASE_RES) * scalev).astype(jnp.int32).astype(jnp.float32)
        row0 = l * (_T // 4)

        def idx_group(g, carry):
            s0 = g * 16
            hs = []
            for dd in range(d):
                x16 = xcols[dd, pl.ds(s0, 16)]
                pos = x16 * resv
                c0i = pos.astype(jnp.int32)
                frac = pos - c0i.astype(jnp.float32)
                fracb[dd, pl.ds(s0, 16)] = frac
                c0u = plsc.bitcast(c0i, jnp.uint32)
                h0 = c0u * _PRIMES[dd] if dd else c0u
                h1 = h0 + _PRIMES[dd]
                hs.append((h0, h1))
            for c in range(ncorner):
                h = hs[0][c & 1]
                for dd in range(1, d):
                    h = h ^ hs[dd][(c >> dd) & 1]
                t = plsc.bitcast(h & _MASK, jnp.int32)
                r0 = (row0 + lax.shift_right_logical(t, 7) * 32
                      + (lax.shift_right_logical(t, 3) & 15))
                idxb[pl.ds(2 * c * _B + s0, 16)] = r0
                idxb[pl.ds((2 * c + 1) * _B + s0, 16)] = r0 + 16
                subb[pl.ds(c * _B + s0, 16)] = t & 7
            return carry

        lax.fori_loop(0, _G, idx_group, None)
        return pltpu.async_copy(tbl_ref.at[idxb], rowsb, sem)

    def acc_phase(l, d, encb, bufs):
        fracb, idxb, subb, rowsb, sem = bufs
        ncorner = 1 << d

        def acc_group(g, carry):
            s0 = g * 16
            svec = s0 + iota16
            fg = []
            for dd in range(d):
                f = fracb[dd, pl.ds(s0, 16)]
                fg.append((1.0 - f, f))
            acc0 = jnp.zeros((16,), jnp.float32)
            acc1 = jnp.zeros((16,), jnp.float32)
            for c in range(ncorner):
                w = fg[0][c & 1]
                for dd in range(1, d):
                    w = w * fg[dd][(c >> dd) & 1]
                sub = subb[pl.ds(c * _B + s0, 16)]
                f0 = plsc.load_gather(rowsb, [2 * c * _B + svec, sub])
                f1 = plsc.load_gather(rowsb, [(2 * c + 1) * _B + svec, sub])
                acc0 = acc0 + w * f0
                acc1 = acc1 + w * f1
            colv = jnp.full((16,), 2 * l, jnp.int32)
            plsc.store_scatter(encb, [svec, colv], acc0)
            plsc.store_scatter(encb, [svec, colv + 1], acc1)
            return carry

        lax.fori_loop(0, _G, acc_group, None)

    ones_f = jnp.ones((16,), jnp.float32)

    def chunk(i, carry):
        base = wid * (_B * _CHUNKS) + i * _B
        pltpu.sync_copy(xyztT.at[wid * _CHUNKS + i], xcols)

        def pair_body(d, tbl_ref, encb, scale, bufsA, bufsB):
            def f(p, sv):
                l0 = 2 * p
                sv1 = sv * jnp.float32(scale)
                dscA = idx_phase(l0, sv, d, tbl_ref, bufsA)
                dscB = idx_phase(l0 + 1, sv1, d, tbl_ref, bufsB)
                dscA.wait()
                acc_phase(l0, d, encb, bufsA)
                dscB.wait()
                acc_phase(l0 + 1, d, encb, bufsB)
                return sv1 * jnp.float32(scale)
            return f

        lax.fori_loop(0, _PROB_LEVELS // 2,
                      pair_body(3, ptbl, encPb, 1.5, set3a, set3b), ones_f)
        pltpu.sync_copy(encPb, encP.at[pl.ds(base, _B)])

        lax.fori_loop(0, _L3D // 2,
                      pair_body(3, stbl, encSb, 1.3819, set3a, set3b), ones_f)
        pltpu.sync_copy(encSb, encS.at[pl.ds(base, _B)])

        lax.fori_loop(0, _L4D // 2,
                      pair_body(4, dtbl, encDb, 1.3819, set4a, set4b), ones_f)
        pltpu.sync_copy(encDb, encD.at[pl.ds(base, _B)])
        return carry

    lax.fori_loop(0, _CHUNKS, chunk, None)


def _sc_encode(xyztT, ptbl, stbl, dtbl):
    mesh = plsc.VectorSubcoreMesh(core_axis_name="c", subcore_axis_name="s")
    f32 = jnp.float32
    return pl.kernel(
        _sc_encode_body,
        out_type=[
            jax.ShapeDtypeStruct((_N, 2 * _PROB_LEVELS), f32),
            jax.ShapeDtypeStruct((_N, 2 * _L3D), f32),
            jax.ShapeDtypeStruct((_N, 2 * _L4D), f32),
        ],
        mesh=mesh,
        compiler_params=pltpu.CompilerParams(needs_layout_passes=False,
                                             use_tc_tiling_on_sc=False),
        scratch_types=(
            [pltpu.VMEM((4, _B), f32)]
            + 2 * [
                pltpu.VMEM((4, _B), f32),
                pltpu.VMEM((16 * _B,), jnp.int32),
                pltpu.VMEM((8 * _B,), jnp.int32),
                pltpu.VMEM((16 * _B, 8), f32),
            ]
            + 2 * [
                pltpu.VMEM((4, _B), f32),
                pltpu.VMEM((32 * _B,), jnp.int32),
                pltpu.VMEM((16 * _B,), jnp.int32),
                pltpu.VMEM((32 * _B, 8), f32),
            ]
            + [
                pltpu.VMEM((_B, 2 * _PROB_LEVELS), f32),
                pltpu.VMEM((_B, 2 * _L3D), f32),
                pltpu.VMEM((_B, 2 * _L4D), f32),
                pltpu.SemaphoreType.DMA,
                pltpu.SemaphoreType.DMA,
            ]
        ),
    )(xyztT, ptbl, stbl, dtbl)


_BLK = 2048


def _mlp_gate_body(encP, encS, encD, pW1, pW2, pW3, sW1, sW2, sW3,
                   dW1, dW2, dW3, o_p, o_s, o_d, o_gs, o_gd, o_w):
    def mm(a, b):
        return lax.dot_general(a, b, (((1,), (0,)), ((), ())),
                               precision=lax.Precision.HIGHEST,
                               preferred_element_type=jnp.float32)

    def mlp(e, W1, W2, W3):
        h = jnp.maximum(mm(e, W1[...]), 0.0)
        h = jnp.maximum(mm(h, W2[...]), 0.0)
        return mm(h, W3[...])

    p = mlp(encP[...], pW1, pW2, pW3)
    s = mlp(encS[...], sW1, sW2, sW3)
    d = mlp(encD[...], dW1, dW2, dW3)
    gs = (1.0 - p) * s
    gd = p * d
    o_p[...] = p
    o_s[...] = s
    o_d[...] = d
    o_gs[...] = gs
    o_gd[...] = gd
    o_w[...] = gs + gd


def _mlp_gate(encP, encS, encD, weights):
    f32 = jnp.float32
    full = lambda shape: pl.BlockSpec(shape, lambda i: (0, 0))
    out_spec = pl.BlockSpec((_BLK, 1), lambda i: (i, 0))
    return pl.pallas_call(
        _mlp_gate_body,
        grid=(_N // _BLK,),
        in_specs=[
            pl.BlockSpec((_BLK, 2 * _PROB_LEVELS), lambda i: (i, 0)),
            pl.BlockSpec((_BLK, 2 * _L3D), lambda i: (i, 0)),
            pl.BlockSpec((_BLK, 2 * _L4D), lambda i: (i, 0)),
            full((2 * _PROB_LEVELS, 64)), full((64, 64)), full((64, 1)),
            full((2 * _L3D, 64)), full((64, 64)), full((64, 1)),
            full((2 * _L4D, 64)), full((64, 64)), full((64, 1)),
        ],
        out_specs=[out_spec] * 6,
        out_shape=[jax.ShapeDtypeStruct((_N, 1), f32)] * 6,
    )(encP, encS, encD, *weights)


def kernel(xyzt, prob_table, table3d, table4d,
           pW1, pW2, pW3, sW1, sW2, sW3, dW1, dW2, dW3):
    xyztT = xyzt.reshape(_N // 128, 128, 4).swapaxes(1, 2)

    def tview(tbl, L):
        return (tbl.reshape(L, _T // 128, 128, 2).swapaxes(2, 3)
                .reshape(L * _T // 4, 8))

    ptbl = tview(prob_table, _PROB_LEVELS)
    stbl = tview(table3d, _L3D)
    dtbl = tview(table4d, _L4D)
    encP, encS, encD = _sc_encode(xyztT, ptbl, stbl, dtbl)
    return _mlp_gate(encP, encS, encD,
                     (pW1, pW2, pW3, sW1, sW2, sW3, dW1, dW2, dW3))


# speedup vs baseline: 92.5659x; 1.0197x over previous
import numpy as np
import jax
import jax.numpy as jnp
from jax import lax
from jax.experimental import pallas as pl
from jax.experimental.pallas import tpu as pltpu
from jax.experimental.pallas import tpu_sc as plsc

_N = 262144
_LOG2_T = 19
_T = 1 << _LOG2_T
_MASK = np.uint32(_T - 1)
_PRIMES = (np.uint32(1), np.uint32(2654435761), np.uint32(805459861),
           np.uint32(3674653429))
_BASE_RES = 16
_PROB_LEVELS = 8
_L3D = 16
_L4D = 16

_NC, _NS = 2, 16
_NW = _NC * _NS
_B = 128
_G = _B // 16
_CHUNKS = _N // (_NW * _B)


def _sc_encode_body(xyztT, ptbl, stbl, dtbl,
                    encP, encS, encD,
                    xcols,
                    frac3a, idx3a, sub3a, rows3a0, rows3a1,
                    frac3b, idx3b, sub3b, rows3b0, rows3b1,
                    frac4a, idx4a, sub4a, rows4a0, rows4a1,
                    frac4b, idx4b, sub4b, rows4b0, rows4b1,
                    encPb, encSb, encDb, semA, semB):
    wid = lax.axis_index("s") * _NC + lax.axis_index("c")
    iota16 = lax.iota(jnp.int32, 16)

    set3a = (frac3a, idx3a, sub3a, rows3a0, rows3a1, semA)
    set3b = (frac3b, idx3b, sub3b, rows3b0, rows3b1, semB)
    set4a = (frac4a, idx4a, sub4a, rows4a0, rows4a1, semA)
    set4b = (frac4b, idx4b, sub4b, rows4b0, rows4b1, semB)

    def idx_phase(l, scalev, d, tbl_ref, bufs):
        fracb, idxb, subb, rows0, rows1, sem = bufs
        ncorner = 1 << d
        resv = (jnp.float32(_BASE_RES) * scalev).astype(jnp.int32).astype(jnp.float32)
        row0 = l * (_T // 4)

        def idx_group(g, carry):
            s0 = g * 16
            hs = []
            for dd in range(d):
                x16 = xcols[dd, pl.ds(s0, 16)]
                pos = x16 * resv
                c0i = pos.astype(jnp.int32)
                frac = pos - c0i.astype(jnp.float32)
                fracb[dd, pl.ds(s0, 16)] = frac
                c0u = plsc.bitcast(c0i, jnp.uint32)
                h0 = c0u * _PRIMES[dd] if dd else c0u
                h1 = h0 + _PRIMES[dd]
                hs.append((h0, h1))
            for c in range(ncorner):
                h = hs[0][c & 1]
                for dd in range(1, d):
                    h = h ^ hs[dd][(c >> dd) & 1]
                t = plsc.bitcast(h & _MASK, jnp.int32)
                r0 = (row0 + lax.shift_right_logical(t, 3)
                      + lax.shift_right_logical(t, 7) * 16)
                idxb[pl.ds(c * _B + s0, 16)] = r0
                subb[pl.ds(c * _B + s0, 16)] = t & 7
            return carry

        lax.fori_loop(0, _G, idx_group, None)
        tbl_f1 = tbl_ref.at[pl.ds(16, tbl_ref.shape[0] - 16)]
        pltpu.async_copy(tbl_ref.at[idxb], rows0, sem)
        pltpu.async_copy(tbl_f1.at[idxb], rows1, sem)

    def wait2(tbl_ref, bufs):
        fracb, idxb, subb, rows0, rows1, sem = bufs
        tbl_f1 = tbl_ref.at[pl.ds(16, tbl_ref.shape[0] - 16)]
        pltpu.make_async_copy(tbl_ref.at[idxb], rows0, sem).wait()
        pltpu.make_async_copy(tbl_f1.at[idxb], rows1, sem).wait()

    def acc_phase(l, d, encb, bufs):
        fracb, idxb, subb, rows0, rows1, sem = bufs
        ncorner = 1 << d

        def acc_group(g, carry):
            s0 = g * 16
            svec = s0 + iota16
            fg = []
            for dd in range(d):
                f = fracb[dd, pl.ds(s0, 16)]
                fg.append((1.0 - f, f))
            acc0 = jnp.zeros((16,), jnp.float32)
            acc1 = jnp.zeros((16,), jnp.float32)
            for c in range(ncorner):
                w = fg[0][c & 1]
                for dd in range(1, d):
                    w = w * fg[dd][(c >> dd) & 1]
                rvec = c * _B + svec
                sub = subb[pl.ds(c * _B + s0, 16)]
                f0 = plsc.load_gather(rows0, [rvec, sub])
                f1 = plsc.load_gather(rows1, [rvec, sub])
                acc0 = acc0 + w * f0
                acc1 = acc1 + w * f1
            colv = jnp.full((16,), 2 * l, jnp.int32)
            plsc.store_scatter(encb, [svec, colv], acc0)
            plsc.store_scatter(encb, [svec, colv + 1], acc1)
            return carry

        lax.fori_loop(0, _G, acc_group, None)

    ones_f = jnp.ones((16,), jnp.float32)

    def chunk(i, carry):
        base = wid * (_B * _CHUNKS) + i * _B
        pltpu.sync_copy(xyztT.at[wid * _CHUNKS + i], xcols)

        def run_encoding(d, tbl_ref, encb, scale, bufsA, bufsB, L):
            idx_phase(0, ones_f, d, tbl_ref, bufsA)
            npairs = L // 2

            def f(p, sv):
                l0 = 2 * p
                sv1 = sv * jnp.float32(scale)
                sv2 = sv1 * jnp.float32(scale)
                idx_phase(l0 + 1, sv1, d, tbl_ref, bufsB)
                wait2(tbl_ref, bufsA)
                acc_phase(l0, d, encb, bufsA)

                @pl.when(p + 1 < npairs)
                def _():
                    idx_phase(l0 + 2, sv2, d, tbl_ref, bufsA)

                wait2(tbl_ref, bufsB)
                acc_phase(l0 + 1, d, encb, bufsB)
                return sv2

            lax.fori_loop(0, npairs, f, ones_f)

        run_encoding(3, ptbl, encPb, 1.5, set3a, set3b, _PROB_LEVELS)
        pltpu.sync_copy(encPb, encP.at[pl.ds(base, _B)])

        run_encoding(3, stbl, encSb, 1.3819, set3a, set3b, _L3D)
        pltpu.sync_copy(encSb, encS.at[pl.ds(base, _B)])

        run_encoding(4, dtbl, encDb, 1.3819, set4a, set4b, _L4D)
        pltpu.sync_copy(encDb, encD.at[pl.ds(base, _B)])
        return carry

    lax.fori_loop(0, _CHUNKS, chunk, None)


def _sc_encode(xyztT, ptbl, stbl, dtbl):
    mesh = plsc.VectorSubcoreMesh(core_axis_name="c", subcore_axis_name="s")
    f32 = jnp.float32
    return pl.kernel(
        _sc_encode_body,
        out_type=[
            jax.ShapeDtypeStruct((_N, 2 * _PROB_LEVELS), f32),
            jax.ShapeDtypeStruct((_N, 2 * _L3D), f32),
            jax.ShapeDtypeStruct((_N, 2 * _L4D), f32),
        ],
        mesh=mesh,
        compiler_params=pltpu.CompilerParams(needs_layout_passes=False,
                                             use_tc_tiling_on_sc=False),
        scratch_types=(
            [pltpu.VMEM((4, _B), f32)]
            + 2 * [
                pltpu.VMEM((4, _B), f32),
                pltpu.VMEM((8 * _B,), jnp.int32),
                pltpu.VMEM((8 * _B,), jnp.int32),
                pltpu.VMEM((8 * _B, 8), f32),
                pltpu.VMEM((8 * _B, 8), f32),
            ]
            + 2 * [
                pltpu.VMEM((4, _B), f32),
                pltpu.VMEM((16 * _B,), jnp.int32),
                pltpu.VMEM((16 * _B,), jnp.int32),
                pltpu.VMEM((16 * _B, 8), f32),
                pltpu.VMEM((16 * _B, 8), f32),
            ]
            + [
                pltpu.VMEM((_B, 2 * _PROB_LEVELS), f32),
                pltpu.VMEM((_B, 2 * _L3D), f32),
                pltpu.VMEM((_B, 2 * _L4D), f32),
                pltpu.SemaphoreType.DMA,
                pltpu.SemaphoreType.DMA,
            ]
        ),
    )(xyztT, ptbl, stbl, dtbl)


_BLK = 2048


def _mlp_gate_body(encP, encS, encD, pW1, pW2, pW3, sW1, sW2, sW3,
                   dW1, dW2, dW3, o_p, o_s, o_d, o_gs, o_gd, o_w):
    def mm(a, b):
        return lax.dot_general(a, b, (((1,), (0,)), ((), ())),
                               precision=lax.Precision.HIGHEST,
                               preferred_element_type=jnp.float32)

    def mlp(e, W1, W2, W3):
        h = jnp.maximum(mm(e, W1[...]), 0.0)
        h = jnp.maximum(mm(h, W2[...]), 0.0)
        return mm(h, W3[...])

    p = mlp(encP[...], pW1, pW2, pW3)
    s = mlp(encS[...], sW1, sW2, sW3)
    d = mlp(encD[...], dW1, dW2, dW3)
    gs = (1.0 - p) * s
    gd = p * d
    o_p[...] = p
    o_s[...] = s
    o_d[...] = d
    o_gs[...] = gs
    o_gd[...] = gd
    o_w[...] = gs + gd


def _mlp_gate(encP, encS, encD, weights):
    f32 = jnp.float32
    full = lambda shape: pl.BlockSpec(shape, lambda i: (0, 0))
    out_spec = pl.BlockSpec((_BLK, 1), lambda i: (i, 0))
    return pl.pallas_call(
        _mlp_gate_body,
        grid=(_N // _BLK,),
        in_specs=[
            pl.BlockSpec((_BLK, 2 * _PROB_LEVELS), lambda i: (i, 0)),
            pl.BlockSpec((_BLK, 2 * _L3D), lambda i: (i, 0)),
            pl.BlockSpec((_BLK, 2 * _L4D), lambda i: (i, 0)),
            full((2 * _PROB_LEVELS, 64)), full((64, 64)), full((64, 1)),
            full((2 * _L3D, 64)), full((64, 64)), full((64, 1)),
            full((2 * _L4D, 64)), full((64, 64)), full((64, 1)),
        ],
        out_specs=[out_spec] * 6,
        out_shape=[jax.ShapeDtypeStruct((_N, 1), f32)] * 6,
    )(encP, encS, encD, *weights)


def kernel(xyzt, prob_table, table3d, table4d,
           pW1, pW2, pW3, sW1, sW2, sW3, dW1, dW2, dW3):
    xyztT = xyzt.reshape(_N // 128, 128, 4).swapaxes(1, 2)

    def tview(tbl, L):
        return (tbl.reshape(L, _T // 128, 128, 2).swapaxes(2, 3)
                .reshape(L * _T // 4, 8))

    ptbl = tview(prob_table, _PROB_LEVELS)
    stbl = tview(table3d, _L3D)
    dtbl = tview(table4d, _L4D)
    encP, encS, encD = _sc_encode(xyztT, ptbl, stbl, dtbl)
    return _mlp_gate(encP, encS, encD,
                     (pW1, pW2, pW3, sW1, sW2, sW3, dW1, dW2, dW3))


# speedup vs baseline: 94.9990x; 1.0263x over previous
import numpy as np
import jax
import jax.numpy as jnp
from jax import lax
from jax.experimental import pallas as pl
from jax.experimental.pallas import tpu as pltpu
from jax.experimental.pallas import tpu_sc as plsc

_N = 262144
_LOG2_T = 19
_T = 1 << _LOG2_T
_MASK = np.uint32(_T - 1)
_PRIMES = (np.uint32(1), np.uint32(2654435761), np.uint32(805459861),
           np.uint32(3674653429))
_BASE_RES = 16
_PROB_LEVELS = 8
_L3D = 16
_L4D = 16

_NC, _NS = 2, 16
_NW = _NC * _NS
_B = 128
_G = _B // 16
_CHUNKS = _N // (_NW * _B)


def _sc_encode_body(xyztT, ptbl, stbl, dtbl,
                    encP, encS, encD,
                    xcols,
                    frac3a, idx3a, sub3a, rows3a0, rows3a1,
                    frac3b, idx3b, sub3b, rows3b0, rows3b1,
                    frac4a, idx4a, sub4a, rows4a0, rows4a1,
                    frac4b, idx4b, sub4b, rows4b0, rows4b1,
                    encPb, encSb, encDb, semA, semB):
    wid = lax.axis_index("s") * _NC + lax.axis_index("c")
    iota16 = lax.iota(jnp.int32, 16)

    set3a = (frac3a, idx3a, sub3a, rows3a0, rows3a1, semA)
    set3b = (frac3b, idx3b, sub3b, rows3b0, rows3b1, semB)
    set4a = (frac4a, idx4a, sub4a, rows4a0, rows4a1, semA)
    set4b = (frac4b, idx4b, sub4b, rows4b0, rows4b1, semB)

    def idx_phase(l, scalev, d, tbl_ref, bufs):
        fracb, idxb, subb, rows0, rows1, sem = bufs
        ncorner = 1 << d
        resv = (jnp.float32(_BASE_RES) * scalev).astype(jnp.int32).astype(jnp.float32)
        row0 = l * (_T // 4)

        def idx_group(g, carry):
            s0 = g * 16
            hs = []
            for dd in range(d):
                x16 = xcols[dd, pl.ds(s0, 16)]
                pos = x16 * resv
                c0i = pos.astype(jnp.int32)
                frac = pos - c0i.astype(jnp.float32)
                fracb[dd, pl.ds(s0, 16)] = frac
                c0u = plsc.bitcast(c0i, jnp.uint32)
                h0 = c0u * _PRIMES[dd] if dd else c0u
                h1 = h0 + _PRIMES[dd]
                hs.append((h0, h1))
            for c in range(ncorner):
                h = hs[0][c & 1]
                for dd in range(1, d):
                    h = h ^ hs[dd][(c >> dd) & 1]
                t = plsc.bitcast(h & _MASK, jnp.int32)
                r0 = (row0 + lax.shift_right_logical(t, 3)
                      + lax.shift_right_logical(t, 7) * 16)
                idxb[pl.ds(c * _B + s0, 16)] = r0
                subb[pl.ds(c * _B + s0, 16)] = t & 7
            return carry

        lax.fori_loop(0, _G, idx_group, None)
        tbl_f1 = tbl_ref.at[pl.ds(16, tbl_ref.shape[0] - 16)]
        pltpu.async_copy(tbl_ref.at[idxb], rows0, sem)
        pltpu.async_copy(tbl_f1.at[idxb], rows1, sem)

    def wait2(tbl_ref, bufs):
        fracb, idxb, subb, rows0, rows1, sem = bufs
        tbl_f1 = tbl_ref.at[pl.ds(16, tbl_ref.shape[0] - 16)]
        pltpu.make_async_copy(tbl_ref.at[idxb], rows0, sem).wait()
        pltpu.make_async_copy(tbl_f1.at[idxb], rows1, sem).wait()

    def acc_phase(l, d, encb, bufs):
        fracb, idxb, subb, rows0, rows1, sem = bufs
        ncorner = 1 << d

        def acc_group(g, carry):
            s0 = g * 16
            svec = s0 + iota16
            fg = []
            for dd in range(d):
                f = fracb[dd, pl.ds(s0, 16)]
                fg.append((1.0 - f, f))
            wxy = [fg[0][i] * fg[1][j] for j in (0, 1) for i in (0, 1)]
            if d == 3:
                whi = fg[2]
            else:
                whi = [fg[2][i] * fg[3][j] for j in (0, 1) for i in (0, 1)]
            acc0 = jnp.zeros((16,), jnp.float32)
            acc1 = jnp.zeros((16,), jnp.float32)
            for c in range(ncorner):
                w = wxy[c & 3] * whi[c >> 2]
                rvec = c * _B + svec
                sub = subb[pl.ds(c * _B + s0, 16)]
                f0 = plsc.load_gather(rows0, [rvec, sub])
                f1 = plsc.load_gather(rows1, [rvec, sub])
                acc0 = acc0 + w * f0
                acc1 = acc1 + w * f1
            colv = jnp.full((16,), 2 * l, jnp.int32)
            plsc.store_scatter(encb, [svec, colv], acc0)
            plsc.store_scatter(encb, [svec, colv + 1], acc1)
            return carry

        lax.fori_loop(0, _G, acc_group, None)

    ones_f = jnp.ones((16,), jnp.float32)

    def chunk(i, carry):
        base = wid * (_B * _CHUNKS) + i * _B
        pltpu.sync_copy(xyztT.at[wid * _CHUNKS + i], xcols)

        def run_encoding(d, tbl_ref, encb, scale, bufsA, bufsB, L):
            idx_phase(0, ones_f, d, tbl_ref, bufsA)
            npairs = L // 2

            def f(p, sv):
                l0 = 2 * p
                sv1 = sv * jnp.float32(scale)
                sv2 = sv1 * jnp.float32(scale)
                idx_phase(l0 + 1, sv1, d, tbl_ref, bufsB)
                wait2(tbl_ref, bufsA)
                acc_phase(l0, d, encb, bufsA)

                @pl.when(p + 1 < npairs)
                def _():
                    idx_phase(l0 + 2, sv2, d, tbl_ref, bufsA)

                wait2(tbl_ref, bufsB)
                acc_phase(l0 + 1, d, encb, bufsB)
                return sv2

            lax.fori_loop(0, npairs, f, ones_f)

        run_encoding(3, ptbl, encPb, 1.5, set3a, set3b, _PROB_LEVELS)
        pltpu.sync_copy(encPb, encP.at[pl.ds(base, _B)])

        run_encoding(3, stbl, encSb, 1.3819, set3a, set3b, _L3D)
        pltpu.sync_copy(encSb, encS.at[pl.ds(base, _B)])

        run_encoding(4, dtbl, encDb, 1.3819, set4a, set4b, _L4D)
        pltpu.sync_copy(encDb, encD.at[pl.ds(base, _B)])
        return carry

    lax.fori_loop(0, _CHUNKS, chunk, None)


def _sc_encode(xyztT, ptbl, stbl, dtbl):
    mesh = plsc.VectorSubcoreMesh(core_axis_name="c", subcore_axis_name="s")
    f32 = jnp.float32
    return pl.kernel(
        _sc_encode_body,
        out_type=[
            jax.ShapeDtypeStruct((_N, 2 * _PROB_LEVELS), f32),
            jax.ShapeDtypeStruct((_N, 2 * _L3D), f32),
            jax.ShapeDtypeStruct((_N, 2 * _L4D), f32),
        ],
        mesh=mesh,
        compiler_params=pltpu.CompilerParams(needs_layout_passes=False,
                                             use_tc_tiling_on_sc=False),
        scratch_types=(
            [pltpu.VMEM((4, _B), f32)]
            + 2 * [
                pltpu.VMEM((4, _B), f32),
                pltpu.VMEM((8 * _B,), jnp.int32),
                pltpu.VMEM((8 * _B,), jnp.int32),
                pltpu.VMEM((8 * _B, 8), f32),
                pltpu.VMEM((8 * _B, 8), f32),
            ]
            + 2 * [
                pltpu.VMEM((4, _B), f32),
                pltpu.VMEM((16 * _B,), jnp.int32),
                pltpu.VMEM((16 * _B,), jnp.int32),
                pltpu.VMEM((16 * _B, 8), f32),
                pltpu.VMEM((16 * _B, 8), f32),
            ]
            + [
                pltpu.VMEM((_B, 2 * _PROB_LEVELS), f32),
                pltpu.VMEM((_B, 2 * _L3D), f32),
                pltpu.VMEM((_B, 2 * _L4D), f32),
                pltpu.SemaphoreType.DMA,
                pltpu.SemaphoreType.DMA,
            ]
        ),
    )(xyztT, ptbl, stbl, dtbl)


_BLK = 2048


def _mlp_gate_body(encP, encS, encD, pW1, pW2, pW3, sW1, sW2, sW3,
                   dW1, dW2, dW3, o_p, o_s, o_d, o_gs, o_gd, o_w):
    def mm(a, b):
        return lax.dot_general(a, b, (((1,), (0,)), ((), ())),
                               precision=lax.Precision.HIGHEST,
                               preferred_element_type=jnp.float32)

    def mlp(e, W1, W2, W3):
        h = jnp.maximum(mm(e, W1[...]), 0.0)
        h = jnp.maximum(mm(h, W2[...]), 0.0)
        return mm(h, W3[...])

    p = mlp(encP[...], pW1, pW2, pW3)
    s = mlp(encS[...], sW1, sW2, sW3)
    d = mlp(encD[...], dW1, dW2, dW3)
    gs = (1.0 - p) * s
    gd = p * d
    o_p[...] = p
    o_s[...] = s
    o_d[...] = d
    o_gs[...] = gs
    o_gd[...] = gd
    o_w[...] = gs + gd


def _mlp_gate(encP, encS, encD, weights):
    f32 = jnp.float32
    full = lambda shape: pl.BlockSpec(shape, lambda i: (0, 0))
    wvec = pl.BlockSpec((64,), lambda i: (0,))
    out_spec = pl.BlockSpec((_BLK,), lambda i: (i,))
    return pl.pallas_call(
        _mlp_gate_body,
        grid=(_N // _BLK,),
        in_specs=[
            pl.BlockSpec((_BLK, 2 * _PROB_LEVELS), lambda i: (i, 0)),
            pl.BlockSpec((_BLK, 2 * _L3D), lambda i: (i, 0)),
            pl.BlockSpec((_BLK, 2 * _L4D), lambda i: (i, 0)),
            full((2 * _PROB_LEVELS, 64)), full((64, 64)), wvec,
            full((2 * _L3D, 64)), full((64, 64)), wvec,
            full((2 * _L4D, 64)), full((64, 64)), wvec,
        ],
        out_specs=[out_spec] * 6,
        out_shape=[jax.ShapeDtypeStruct((_N,), f32)] * 6,
    )(encP, encS, encD, *weights)


def kernel(xyzt, prob_table, table3d, table4d,
           pW1, pW2, pW3, sW1, sW2, sW3, dW1, dW2, dW3):
    xyztT = xyzt.reshape(_N // 128, 128, 4).swapaxes(1, 2)

    def tview(tbl, L):
        return (tbl.reshape(L, _T // 128, 128, 2).swapaxes(2, 3)
                .reshape(L * _T // 4, 8))

    ptbl = tview(prob_table, _PROB_LEVELS)
    stbl = tview(table3d, _L3D)
    dtbl = tview(table4d, _L4D)
    encP, encS, encD = _sc_encode(xyztT, ptbl, stbl, dtbl)
    outs = _mlp_gate(encP, encS, encD,
                     (pW1, pW2, pW3.reshape(64), sW1, sW2, sW3.reshape(64),
                      dW1, dW2, dW3.reshape(64)))
    return tuple(o.reshape(_N, 1) for o in outs)


# speedup vs baseline: 101.6383x; 1.0699x over previous
import numpy as np
import jax
import jax.numpy as jnp
from jax import lax
from jax.experimental import pallas as pl
from jax.experimental.pallas import tpu as pltpu
from jax.experimental.pallas import tpu_sc as plsc

_N = 262144
_LOG2_T = 19
_T = 1 << _LOG2_T
_MASK = np.uint32(_T - 1)
_PRIMES = (np.uint32(1), np.uint32(2654435761), np.uint32(805459861),
           np.uint32(3674653429))
_BASE_RES = 16
_PROB_LEVELS = 8
_L3D = 16
_L4D = 16

_NC, _NS = 2, 16
_NW = _NC * _NS
_B = 128
_G = _B // 16
_CHUNKS = _N // (_NW * _B)


def _sc_encode_body(xyztT, ptbl, stbl, dtbl,
                    encP, encS, encD,
                    xcols,
                    frac3a, idx3a, sub3a, rows3a0, rows3a1,
                    frac3b, idx3b, sub3b, rows3b0, rows3b1,
                    frac4a, idx4a, sub4a, rows4a0, rows4a1,
                    frac4b, idx4b, sub4b, rows4b0, rows4b1,
                    encPb, encSb, encDb, semA, semB):
    wid = lax.axis_index("s") * _NC + lax.axis_index("c")
    iota16 = lax.iota(jnp.int32, 16)

    set3a = (frac3a, idx3a, sub3a, rows3a0, rows3a1, semA)
    set3b = (frac3b, idx3b, sub3b, rows3b0, rows3b1, semB)
    set4a = (frac4a, idx4a, sub4a, rows4a0, rows4a1, semA)
    set4b = (frac4b, idx4b, sub4b, rows4b0, rows4b1, semB)

    def idx_phase(l, scalev, d, tbl_ref, bufs):
        fracb, idxb, subb, rows0, rows1, sem = bufs
        ncorner = 1 << d
        resv = (jnp.float32(_BASE_RES) * scalev).astype(jnp.int32).astype(jnp.float32)
        row0 = l * (_T // 4)

        def idx_group(g, carry):
            s0 = g * 16
            hs = []
            for dd in range(d):
                x16 = xcols[dd, pl.ds(s0, 16)]
                pos = x16 * resv
                c0i = pos.astype(jnp.int32)
                frac = pos - c0i.astype(jnp.float32)
                fracb[dd, pl.ds(s0, 16)] = frac
                c0u = plsc.bitcast(c0i, jnp.uint32)
                h0 = c0u * _PRIMES[dd] if dd else c0u
                h1 = h0 + _PRIMES[dd]
                hs.append((h0, h1))
            for c in range(ncorner):
                h = hs[0][c & 1]
                for dd in range(1, d):
                    h = h ^ hs[dd][(c >> dd) & 1]
                t = plsc.bitcast(h & _MASK, jnp.int32)
                r0 = (row0 + lax.shift_right_logical(t, 3)
                      + lax.shift_right_logical(t, 7) * 16)
                idxb[pl.ds(c * _B + s0, 16)] = r0
                subb[pl.ds(c * _B + s0, 16)] = t & 7
            return carry

        lax.fori_loop(0, _G, idx_group, None, unroll=2)
        tbl_f1 = tbl_ref.at[pl.ds(16, tbl_ref.shape[0] - 16)]
        pltpu.async_copy(tbl_ref.at[idxb], rows0, sem)
        pltpu.async_copy(tbl_f1.at[idxb], rows1, sem)

    def wait2(tbl_ref, bufs):
        fracb, idxb, subb, rows0, rows1, sem = bufs
        tbl_f1 = tbl_ref.at[pl.ds(16, tbl_ref.shape[0] - 16)]
        pltpu.make_async_copy(tbl_ref.at[idxb], rows0, sem).wait()
        pltpu.make_async_copy(tbl_f1.at[idxb], rows1, sem).wait()

    def acc_phase(l, d, encb, bufs):
        fracb, idxb, subb, rows0, rows1, sem = bufs
        ncorner = 1 << d

        def acc_group(g, carry):
            s0 = g * 16
            svec = s0 + iota16
            fg = []
            for dd in range(d):
                f = fracb[dd, pl.ds(s0, 16)]
                fg.append((1.0 - f, f))
            wxy = [fg[0][i] * fg[1][j] for j in (0, 1) for i in (0, 1)]
            if d == 3:
                whi = fg[2]
            else:
                whi = [fg[2][i] * fg[3][j] for j in (0, 1) for i in (0, 1)]
            acc0 = jnp.zeros((16,), jnp.float32)
            acc1 = jnp.zeros((16,), jnp.float32)
            for c in range(ncorner):
                w = wxy[c & 3] * whi[c >> 2]
                rvec = c * _B + svec
                sub = subb[pl.ds(c * _B + s0, 16)]
                f0 = plsc.load_gather(rows0, [rvec, sub])
                f1 = plsc.load_gather(rows1, [rvec, sub])
                acc0 = acc0 + w * f0
                acc1 = acc1 + w * f1
            colv = jnp.full((16,), 2 * l, jnp.int32)
            plsc.store_scatter(encb, [svec, colv], acc0)
            plsc.store_scatter(encb, [svec, colv + 1], acc1)
            return carry

        lax.fori_loop(0, _G, acc_group, None, unroll=2)

    ones_f = jnp.ones((16,), jnp.float32)

    def chunk(i, carry):
        base = wid * (_B * _CHUNKS) + i * _B
        pltpu.sync_copy(xyztT.at[wid * _CHUNKS + i], xcols)

        def run_encoding(d, tbl_ref, encb, scale, bufsA, bufsB, L):
            idx_phase(0, ones_f, d, tbl_ref, bufsA)
            npairs = L // 2

            def f(p, sv):
                l0 = 2 * p
                sv1 = sv * jnp.float32(scale)
                sv2 = sv1 * jnp.float32(scale)
                idx_phase(l0 + 1, sv1, d, tbl_ref, bufsB)
                wait2(tbl_ref, bufsA)
                acc_phase(l0, d, encb, bufsA)

                @pl.when(p + 1 < npairs)
                def _():
                    idx_phase(l0 + 2, sv2, d, tbl_ref, bufsA)

                wait2(tbl_ref, bufsB)
                acc_phase(l0 + 1, d, encb, bufsB)
                return sv2

            lax.fori_loop(0, npairs, f, ones_f)

        run_encoding(3, ptbl, encPb, 1.5, set3a, set3b, _PROB_LEVELS)
        pltpu.sync_copy(encPb, encP.at[pl.ds(base, _B)])

        run_encoding(3, stbl, encSb, 1.3819, set3a, set3b, _L3D)
        pltpu.sync_copy(encSb, encS.at[pl.ds(base, _B)])

        run_encoding(4, dtbl, encDb, 1.3819, set4a, set4b, _L4D)
        pltpu.sync_copy(encDb, encD.at[pl.ds(base, _B)])
        return carry

    lax.fori_loop(0, _CHUNKS, chunk, None)


def _sc_encode(xyztT, ptbl, stbl, dtbl):
    mesh = plsc.VectorSubcoreMesh(core_axis_name="c", subcore_axis_name="s")
    f32 = jnp.float32
    return pl.kernel(
        _sc_encode_body,
        out_type=[
            jax.ShapeDtypeStruct((_N, 2 * _PROB_LEVELS), f32),
            jax.ShapeDtypeStruct((_N, 2 * _L3D), f32),
            jax.ShapeDtypeStruct((_N, 2 * _L4D), f32),
        ],
        mesh=mesh,
        compiler_params=pltpu.CompilerParams(needs_layout_passes=False,
                                             use_tc_tiling_on_sc=False),
        scratch_types=(
            [pltpu.VMEM((4, _B), f32)]
            + 2 * [
                pltpu.VMEM((4, _B), f32),
                pltpu.VMEM((8 * _B,), jnp.int32),
                pltpu.VMEM((8 * _B,), jnp.int32),
                pltpu.VMEM((8 * _B, 8), f32),
                pltpu.VMEM((8 * _B, 8), f32),
            ]
            + 2 * [
                pltpu.VMEM((4, _B), f32),
                pltpu.VMEM((16 * _B,), jnp.int32),
                pltpu.VMEM((16 * _B,), jnp.int32),
                pltpu.VMEM((16 * _B, 8), f32),
                pltpu.VMEM((16 * _B, 8), f32),
            ]
            + [
                pltpu.VMEM((_B, 2 * _PROB_LEVELS), f32),
                pltpu.VMEM((_B, 2 * _L3D), f32),
                pltpu.VMEM((_B, 2 * _L4D), f32),
                pltpu.SemaphoreType.DMA,
                pltpu.SemaphoreType.DMA,
            ]
        ),
    )(xyztT, ptbl, stbl, dtbl)


_BLK = 2048


def _mlp_gate_body(encP, encS, encD, pW1, pW2, pW3, sW1, sW2, sW3,
                   dW1, dW2, dW3, o_p, o_s, o_d, o_gs, o_gd, o_w):
    def mm(a, b):
        return lax.dot_general(a, b, (((1,), (0,)), ((), ())),
                               preferred_element_type=jnp.float32)

    def mlp(e, W1, W2, W3):
        h = jnp.maximum(mm(e, W1[...]), 0.0)
        h = jnp.maximum(mm(h, W2[...]), 0.0)
        return mm(h, W3[...])

    p = mlp(encP[...], pW1, pW2, pW3)
    s = mlp(encS[...], sW1, sW2, sW3)
    d = mlp(encD[...], dW1, dW2, dW3)
    gs = (1.0 - p) * s
    gd = p * d
    o_p[...] = p
    o_s[...] = s
    o_d[...] = d
    o_gs[...] = gs
    o_gd[...] = gd
    o_w[...] = gs + gd


def _mlp_gate(encP, encS, encD, weights):
    f32 = jnp.float32
    full = lambda shape: pl.BlockSpec(shape, lambda i: (0, 0))
    wvec = pl.BlockSpec((64,), lambda i: (0,))
    out_spec = pl.BlockSpec((_BLK,), lambda i: (i,))
    return pl.pallas_call(
        _mlp_gate_body,
        grid=(_N // _BLK,),
        in_specs=[
            pl.BlockSpec((_BLK, 2 * _PROB_LEVELS), lambda i: (i, 0)),
            pl.BlockSpec((_BLK, 2 * _L3D), lambda i: (i, 0)),
            pl.BlockSpec((_BLK, 2 * _L4D), lambda i: (i, 0)),
            full((2 * _PROB_LEVELS, 64)), full((64, 64)), wvec,
            full((2 * _L3D, 64)), full((64, 64)), wvec,
            full((2 * _L4D, 64)), full((64, 64)), wvec,
        ],
        out_specs=[out_spec] * 6,
        out_shape=[jax.ShapeDtypeStruct((_N,), f32)] * 6,
    )(encP, encS, encD, *weights)


def kernel(xyzt, prob_table, table3d, table4d,
           pW1, pW2, pW3, sW1, sW2, sW3, dW1, dW2, dW3):
    xyztT = xyzt.reshape(_N // 128, 128, 4).swapaxes(1, 2)

    def tview(tbl, L):
        return (tbl.reshape(L, _T // 128, 128, 2).swapaxes(2, 3)
                .reshape(L * _T // 4, 8))

    ptbl = tview(prob_table, _PROB_LEVELS)
    stbl = tview(table3d, _L3D)
    dtbl = tview(table4d, _L4D)
    encP, encS, encD = _sc_encode(xyztT, ptbl, stbl, dtbl)
    outs = _mlp_gate(encP, encS, encD,
                     (pW1, pW2, pW3.reshape(64), sW1, sW2, sW3.reshape(64),
                      dW1, dW2, dW3.reshape(64)))
    return tuple(o.reshape(_N, 1) for o in outs)
